# baseline (speedup 1.0000x reference)
import functools

import jax
import jax.numpy as jnp
from jax.experimental import pallas as pl
from jax.experimental.pallas import tpu as pltpu

_INF = 1e30


def _dtw_kernel(x_ref, y_ref, yt_ref, cost_ref, wvs_ref, rt_ref, dec_ref,
                last_ref, init_ref, *, B, N, M, d):
    lane2 = jax.lax.broadcasted_iota(jnp.int32, (M, N), 1)

    def build(b, carry):
        xb = x_ref[pl.ds(b, 1)][0]
        yb = y_ref[pl.ds(b, 1)][0]
        G = jax.lax.dot_general(yb, xb, (((1,), (1,)), ((), ())),
                                preferred_element_type=jnp.float32)
        y2 = jnp.sum(yb * yb, axis=1, keepdims=True)
        ones = jnp.ones((1, d), jnp.float32)
        x2 = jax.lax.dot_general(ones, xb * xb, (((1,), (1,)), ((), ())),
                                 preferred_element_type=jnp.float32)
        r = y2 + x2 - 2.0 * G
        for t in range(9):
            m = ((lane2 >> t) & 1) == 1
            r = jnp.where(m, jnp.roll(r, 1 << t, axis=0), r)
        rt_ref[pl.ds(b, 1)] = r[None]
        return carry

    jax.lax.fori_loop(0, B, build, jnp.int32(0))

    lane1 = jax.lax.broadcasted_iota(jnp.int32, (B, N), 1)
    subB = jax.lax.broadcasted_iota(jnp.int32, (B, B), 0)
    laneB = jax.lax.broadcasted_iota(jnp.int32, (B, B), 1)

    init_ref[...] = jnp.full((B, N), _INF, jnp.float32)
    inf_rows = init_ref[...]

    def dp_step(k, carry):
        d1, d2 = carry
        rowidx = k - jnp.where(k >= M, M, 0)
        cdiag = jnp.concatenate(
            [rt_ref[b, pl.ds(rowidx, 1), :] for b in range(B)], axis=0)
        d1s = jnp.where(lane1 == 0, _INF, jnp.roll(d1, 1, axis=1))
        d2s = jnp.where(lane1 == 0, _INF, jnp.roll(d2, 1, axis=1))
        b0 = (d2s <= d1s) & (d2s <= d1)
        b1 = jnp.logical_not(b0) & (d1s <= d1)
        dec = jnp.where(b0, 0, jnp.where(b1, 1, 2)).astype(jnp.int32)
        dec_ref[pl.ds(k * B, B), :] = dec
        new = cdiag + jnp.minimum(jnp.minimum(d1s, d1), d2s)
        new = jnp.where((lane1 == 0) & (k <= M - 1), cdiag, new)
        valid = (lane1 <= k) & (lane1 >= k - (M - 1))
        new = jnp.where(valid, new, _INF)
        lastv = jnp.sum(jnp.where(lane1 == N - 1, new, 0.0), axis=1,
                        keepdims=True)
        lastrow = jnp.sum(jnp.where(subB == laneB, lastv, 0.0), axis=0,
                          keepdims=True)
        last_ref[pl.ds(k, 1), :] = lastrow
        return (new, d1)

    jax.lax.fori_loop(0, N + M - 1, dp_step, (inf_rows, inf_rows))

    lastblk = last_ref[pl.ds(N - 1, M), :]
    mvals = jnp.min(lastblk, axis=0, keepdims=True)
    subM = jax.lax.broadcasted_iota(jnp.int32, (M, B), 0)
    jrow = jnp.min(jnp.where(lastblk == mvals, subM, jnp.int32(1 << 20)),
                   axis=0, keepdims=True)
    laneB1 = jax.lax.broadcasted_iota(jnp.int32, (1, B), 1)
    lane1r = jax.lax.broadcasted_iota(jnp.int32, (1, N), 1)

    def extract(row, idx):
        return jnp.sum(jnp.where(lane1r == idx, row, jnp.float32(0.0)))

    def iextract(row, idx):
        return jnp.sum(jnp.where(lane1r == idx, row, jnp.int32(0)))

    ytrows = []
    init_w = []
    init_j = []
    init_ref[...] = jnp.zeros((B, N), jnp.float32)
    zrows = init_ref[...]
    for b in range(B):
        mval = jnp.sum(jnp.where(laneB1 == b, mvals, jnp.float32(0.0)))
        j_end = jnp.sum(jnp.where(laneB1 == b, jrow, jnp.int32(0)))
        cost_ref[pl.ds(b, 1)] = jnp.full((1, 1, 128), mval)
        ytrow = yt_ref[pl.ds(b, 1), :]
        ytrows.append(ytrow)
        init_j.append(j_end)
        init_w.append(jnp.where(lane1r == N - 1, extract(ytrow, j_end),
                                zrows[b:b + 1, :]))

    def bt_cond(s):
        done_all = s[4 * B]
        return jnp.logical_not(done_all)

    def bt_body(s):
        ii = s[0:B]
        jj = s[B:2 * B]
        dd = s[2 * B:3 * B]
        ww = s[3 * B:4 * B]
        nii, njj, ndd, nww = [], [], [], []
        for b in range(B):
            i, j, done, wv = ii[b], jj[b], dd[b], ww[b]
            k = i + j
            drow = dec_ref[pl.ds(jnp.maximum(k, 0) * B + b, 1), :]
            dcn = iextract(drow, i)
            di = jnp.where(dcn == 2, 0, 1)
            dj = jnp.where(dcn == 1, 0, 1)
            stop = jnp.logical_or(done, i == 0)
            ni = jnp.where(stop, i, i - di)
            nj = jnp.where(stop, j, j - dj)
            idx = jnp.where(ni != i, ni, jnp.int32(-1))
            nww.append(jnp.where(lane1r == idx, extract(ytrows[b], nj), wv))
            nii.append(ni)
            njj.append(nj)
            ndd.append(stop)
        done_all = ndd[0]
        for b in range(1, B):
            done_all = jnp.logical_and(done_all, ndd[b])
        return (tuple(nii) + tuple(njj) + tuple(ndd) + tuple(nww)
                + (done_all,))

    init_i = tuple(jnp.int32(N - 1) for _ in range(B))
    init_d = tuple(jnp.bool_(False) for _ in range(B))
    fin = jax.lax.while_loop(
        bt_cond, bt_body,
        init_i + tuple(init_j) + init_d + tuple(init_w) + (jnp.bool_(False),))
    for b in range(B):
        wvs_ref[pl.ds(b, 1)] = fin[3 * B + b][None]


def _dtw_call(x, y, y_t, interpret=False):
    B, N, d = x.shape
    M = y.shape[1]
    kfn = functools.partial(_dtw_kernel, B=B, N=N, M=M, d=d)
    cost, w_vs = pl.pallas_call(
        kfn,
        in_specs=[
            pl.BlockSpec((B, N, d), lambda: (0, 0, 0)),
            pl.BlockSpec((B, M, d), lambda: (0, 0, 0)),
            pl.BlockSpec((B, M), lambda: (0, 0)),
        ],
        out_specs=[
            pl.BlockSpec((B, 1, 128), lambda: (0, 0, 0)),
            pl.BlockSpec((B, 1, N), lambda: (0, 0, 0)),
        ],
        out_shape=[
            jax.ShapeDtypeStruct((B, 1, 128), jnp.float32),
            jax.ShapeDtypeStruct((B, 1, N), jnp.float32),
        ],
        scratch_shapes=[
            pltpu.VMEM((B, M, N), jnp.float32),
            pltpu.VMEM(((N + M) * B, N), jnp.int32),
            pltpu.VMEM((N + M, B), jnp.float32),
            pltpu.VMEM((B, N), jnp.float32),
        ],
        interpret=interpret,
    )(x, y, y_t)
    return cost, w_vs


def kernel(x, y, x_t, y_t):
    cost, w_vs = _dtw_call(x, y, y_t)
    return cost[:, 0, 0], x_t, w_vs[:, 0, :]
